# baseline (speedup 1.0000x reference)
import functools

import jax
import jax.numpy as jnp
from jax import lax
from jax.experimental import pallas as pl
from jax.experimental.pallas import tpu as pltpu
from jax.experimental.pallas import tpu_sc as plsc

B = 16384
D = 64
NC, NS = 2, 16
NW = NC * NS
B_PER_W = B // NW
H = B_PER_W // 2
G = 16

_MESH = plsc.VectorSubcoreMesh(core_axis_name="c", subcore_axis_name="s")


@functools.partial(
    pl.kernel,
    out_type=jax.ShapeDtypeStruct((B, D), jnp.float32),
    mesh=_MESH,
    scratch_types=[
        pltpu.VMEM((B_PER_W,), jnp.int32),
        pltpu.VMEM((H, D), jnp.float32),
        pltpu.VMEM_SHARED((NS, H, D), jnp.float32),
        pltpu.SemaphoreType.DMA,
        pltpu.SemaphoreType.DMA,
    ],
    compiler_params=pltpu.CompilerParams(use_tc_tiling_on_sc=True),
)
def _gather_impl(idx_hbm, table_hbm, out_hbm, idx_v, rows_v, srows_v, tsem, ssem):
    cid = lax.axis_index("c")
    sid = lax.axis_index("s")
    wid = sid * NC + cid
    base = wid * B_PER_W
    pltpu.sync_copy(idx_hbm.at[pl.ds(base, B_PER_W)], idx_v)

    my_srows = srows_v.at[sid]

    @plsc.parallel_loop(0, H, step=G)
    def _(g):
        vec = idx_v[pl.ds(g, G)]
        svec = idx_v[pl.ds(H + g, G)]
        for l in range(G):
            pltpu.async_copy(
                table_hbm.at[pl.ds(svec[l], 1)],
                my_srows.at[pl.ds(g + l, 1)],
                ssem,
            )
            pltpu.async_copy(
                table_hbm.at[pl.ds(vec[l], 1)],
                rows_v.at[pl.ds(g + l, 1)],
                tsem,
            )

    pltpu.make_async_copy(table_hbm.at[pl.ds(0, H)], rows_v, tsem).wait()
    pltpu.sync_copy(rows_v, out_hbm.at[pl.ds(base, H)])
    pltpu.make_async_copy(table_hbm.at[pl.ds(0, H)], my_srows, ssem).wait()
    pltpu.sync_copy(my_srows, out_hbm.at[pl.ds(base + H, H)])


def kernel(customer_id, table):
    idx = customer_id.astype(jnp.int32)
    return _gather_impl(idx, table)
